# baseline (speedup 1.0000x reference)
import functools

import jax
import jax.numpy as jnp
from jax import lax
from jax.experimental import pallas as pl
from jax.experimental.pallas import tpu as pltpu
from jax.experimental.pallas import tpu_sc as plsc

N = 100000
D = 128
NW = 32
CHUNK = 256
NCH = N // CHUNK
FULL_T = NCH // NW
REM = NCH - FULL_T * NW
TAIL = N - NCH * CHUNK
TAIL_BASE = NCH * CHUNK
TAIL_WID = NW - 1

_mesh = plsc.VectorSubcoreMesh(core_axis_name="c", subcore_axis_name="s")


@functools.partial(
    pl.kernel,
    mesh=_mesh,
    out_type=jax.ShapeDtypeStruct((N, D), jnp.float32),
    scratch_types=[
        pltpu.VMEM((CHUNK,), jnp.int32),
        pltpu.VMEM((CHUNK,), jnp.int32),
        pltpu.VMEM((CHUNK, D), jnp.float32),
        pltpu.VMEM((CHUNK, D), jnp.float32),
        pltpu.VMEM((TAIL,), jnp.int32),
        pltpu.VMEM((TAIL, D), jnp.float32),
        pltpu.SemaphoreType.DMA,
        pltpu.SemaphoreType.DMA,
    ],
)
def _embed_gather(idx_hbm, table_hbm, out_hbm, idx0, idx1, rows0, rows1,
                  idx_t, rows_t, sem_g, sem_s):
    wid = lax.axis_index("s") * 2 + lax.axis_index("c")
    idx_bufs = (idx0, idx1)
    rows_bufs = (rows0, rows1)

    def base(t):
        return (wid + t * NW) * CHUNK

    scat_h = [None] * (FULL_T + 1)

    def do_round(t):
        b = t % 2
        if t >= 2:
            scat_h[t - 2].wait()
        pltpu.sync_copy(idx_hbm.at[pl.ds(base(t), CHUNK)], idx_bufs[b])
        pltpu.async_copy(table_hbm.at[idx_bufs[b]], rows_bufs[b], sem_g).wait()
        scat_h[t] = pltpu.async_copy(
            rows_bufs[b], out_hbm.at[pl.ds(base(t), CHUNK)], sem_s)

    for t in range(FULL_T):
        do_round(t)

    @pl.when(wid < REM)
    def _():
        do_round(FULL_T)

    @pl.when(wid == TAIL_WID)
    def _():
        pltpu.sync_copy(idx_hbm.at[pl.ds(TAIL_BASE, TAIL)], idx_t)
        pltpu.async_copy(table_hbm.at[idx_t], rows_t, sem_g).wait()
        pltpu.async_copy(rows_t, out_hbm.at[pl.ds(TAIL_BASE, TAIL)],
                         sem_s).wait()

    scat_h[FULL_T - 2].wait()
    scat_h[FULL_T - 1].wait()


def kernel(centrality, embedding):
    idx = centrality.astype(jnp.int32)
    return _embed_gather(idx, embedding)


# speedup vs baseline: 10.0556x; 10.0556x over previous
import functools

import jax
import jax.numpy as jnp
from jax import lax
from jax.experimental import pallas as pl
from jax.experimental.pallas import tpu as pltpu
from jax.experimental.pallas import tpu_sc as plsc

N = 100000
D = 128
NW = 32
CHUNK = 256
NCH = N // CHUNK
FULL_T = NCH // NW
REM = NCH - FULL_T * NW
TAIL = N - NCH * CHUNK
TAIL_BASE = NCH * CHUNK
TAIL_WID = NW - 1

_mesh = plsc.VectorSubcoreMesh(core_axis_name="c", subcore_axis_name="s")


@functools.partial(
    pl.kernel,
    mesh=_mesh,
    out_type=jax.ShapeDtypeStruct((N, D), jnp.float32),
    scratch_types=[
        pltpu.VMEM((CHUNK,), jnp.int32),
        pltpu.VMEM((CHUNK,), jnp.int32),
        pltpu.VMEM((CHUNK, D), jnp.float32),
        pltpu.VMEM((CHUNK, D), jnp.float32),
        pltpu.VMEM((TAIL,), jnp.int32),
        pltpu.VMEM((TAIL, D), jnp.float32),
        pltpu.VMEM_SHARED((10, D), jnp.float32),
        pltpu.SemaphoreType.DMA,
        pltpu.SemaphoreType.DMA,
    ],
)
def _embed_gather(idx_hbm, table_hbm, out_hbm, idx0, idx1, rows0, rows1,
                  idx_t, rows_t, table_sh, sem_g, sem_s):
    wid = lax.axis_index("s") * 2 + lax.axis_index("c")
    idx_bufs = (idx0, idx1)
    rows_bufs = (rows0, rows1)

    @pl.when(lax.axis_index("s") == 0)
    def _():
        pltpu.sync_copy(table_hbm, table_sh)
    plsc.subcore_barrier()

    def base(t):
        return (wid + t * NW) * CHUNK

    scat_h = [None] * (FULL_T + 1)

    def do_round(t):
        b = t % 2
        if t >= 2:
            scat_h[t - 2].wait()
        pltpu.sync_copy(idx_hbm.at[pl.ds(base(t), CHUNK)], idx_bufs[b])
        pltpu.async_copy(table_sh.at[idx_bufs[b]], rows_bufs[b], sem_g).wait()
        scat_h[t] = pltpu.async_copy(
            rows_bufs[b], out_hbm.at[pl.ds(base(t), CHUNK)], sem_s)

    for t in range(FULL_T):
        do_round(t)

    @pl.when(wid < REM)
    def _():
        do_round(FULL_T)

    @pl.when(wid == TAIL_WID)
    def _():
        pltpu.sync_copy(idx_hbm.at[pl.ds(TAIL_BASE, TAIL)], idx_t)
        pltpu.async_copy(table_sh.at[idx_t], rows_t, sem_g).wait()
        pltpu.async_copy(rows_t, out_hbm.at[pl.ds(TAIL_BASE, TAIL)],
                         sem_s).wait()

    scat_h[FULL_T - 2].wait()
    scat_h[FULL_T - 1].wait()


def kernel(centrality, embedding):
    idx = centrality.astype(jnp.int32)
    return _embed_gather(idx, embedding)
